# baseline (speedup 1.0000x reference)
import functools

import jax
import jax.numpy as jnp
from jax import lax
from jax.experimental import pallas as pl
from jax.experimental.pallas import tpu as pltpu
from jax.experimental.pallas import tpu_sc as plsc

EMB = 32
SEQ = 2048
NPOS = 1025
HEAD = 1535
NW = 32
ROWS_PER_W = SEQ // NW
WCOLS = SEQ + ROWS_PER_W
ROW_WORDS = SEQ * EMB
CHUNK = 8 * SEQ
TT_W = EMB * NPOS
WT_OFF = TT_W
ARENA = TT_W + EMB * WCOLS


def _sc_call(table_t):
    mesh = plsc.VectorSubcoreMesh(core_axis_name="c", subcore_axis_name="s")

    @functools.partial(
        pl.kernel,
        mesh=mesh,
        out_type=jax.ShapeDtypeStruct((SEQ * ROW_WORDS,), jnp.float32),
        scratch_types=[
            pltpu.VMEM((ARENA,), jnp.float32),
            pltpu.SemaphoreType.DMA,
            pltpu.SemaphoreType.DMA,
            pltpu.SemaphoreType.DMA,
            pltpu.SemaphoreType.DMA,
        ],
        compiler_params=pltpu.CompilerParams(needs_layout_passes=False),
    )
    def body(tt_hbm, out_hbm, arena, sem0, sem1, sem2, sem3):
        w = lax.axis_index("s") * 2 + lax.axis_index("c")
        sems = (sem0, sem1, sem2, sem3)

        pltpu.sync_copy(tt_hbm, arena.at[pl.ds(0, TT_W)])

        u0 = 1984 - ROWS_PER_W * w
        lanes = lax.iota(jnp.int32, 16)

        @plsc.parallel_loop(0, EMB * (WCOLS // 16), step=1, unroll=4)
        def build(k):
            c = k // (WCOLS // 16)
            xv = k % (WCOLS // 16)
            lo = c * NPOS
            base = lo + u0 + 16 * xv - HEAD
            idx = jnp.clip(lanes + base, lo, lo + NPOS - 1)
            vals = plsc.load_gather(arena, [idx])
            arena[pl.ds(WT_OFF + c * WCOLS + 16 * xv, 16)] = vals

        first = w * ROWS_PER_W

        HALF = CHUNK // 2

        def emit_row(t, carry):
            delta = (ROWS_PER_W - 1) - t
            for tr in range(4):
                for h in range(2):
                    b = (tr * 2 + h) % 4

                    def reclaim(b=b):
                        pltpu.make_async_copy(
                            arena.at[pl.ds(b * HALF, HALF)],
                            out_hbm.at[pl.ds(0, HALF)],
                            sems[b],
                        ).wait()

                    if tr < 2:
                        pl.when(t > 0)(reclaim)
                    else:
                        reclaim()

                    @plsc.parallel_loop(64 * h, 64 * h + 64, step=1, unroll=8)
                    def pack(blk):
                        tc = blk // 8
                        r = blk % 8
                        src = (
                            WT_OFF + (8 * tr + r) * WCOLS + delta + 128 * tc
                        )
                        dst = b * HALF + (blk - 64 * h) * 128
                        for q in range(8):
                            arena[pl.ds(dst + 16 * q, 16)] = arena[
                                pl.ds(src + 16 * q, 16)
                            ]

                    out_off = pl.multiple_of(
                        (first + t) * ROW_WORDS + tr * CHUNK + h * HALF,
                        HALF,
                    )
                    pltpu.async_copy(
                        arena.at[pl.ds(b * HALF, HALF)],
                        out_hbm.at[pl.ds(out_off, HALF)],
                        sems[b],
                    )
            return carry

        lax.fori_loop(0, ROWS_PER_W, emit_row, 0)

        for b in range(4):
            pltpu.make_async_copy(
                arena.at[pl.ds(b * HALF, HALF)],
                out_hbm.at[pl.ds(0, HALF)],
                sems[b],
            ).wait()

    return body(table_t)


def kernel(seq_len_q, seq_len_k, embeddings_table):
    del seq_len_q, seq_len_k
    flat = _sc_call(embeddings_table.T.reshape(-1))
    s = flat.reshape(SEQ, 4, 16, 8, 128)
    return s.transpose(0, 2, 4, 1, 3).reshape(SEQ, SEQ, EMB)


# speedup vs baseline: 1.0266x; 1.0266x over previous
import functools

import jax
import jax.numpy as jnp
from jax import lax
from jax.experimental import pallas as pl
from jax.experimental.pallas import tpu as pltpu
from jax.experimental.pallas import tpu_sc as plsc

EMB = 32
SEQ = 2048
NPOS = 1025
HEAD = 1535
NW = 32
ROWS_PER_W = SEQ // NW
WCOLS = SEQ + ROWS_PER_W
ROW_WORDS = SEQ * EMB
CHUNK = 8 * SEQ
TT_W = EMB * NPOS
WT_OFF = TT_W
ARENA = TT_W + EMB * WCOLS


def _sc_call(table_t):
    mesh = plsc.VectorSubcoreMesh(core_axis_name="c", subcore_axis_name="s")

    @functools.partial(
        pl.kernel,
        mesh=mesh,
        out_type=jax.ShapeDtypeStruct((SEQ * ROW_WORDS,), jnp.float32),
        scratch_types=[
            pltpu.VMEM((ARENA,), jnp.float32),
            pltpu.SemaphoreType.DMA,
            pltpu.SemaphoreType.DMA,
        ],
        compiler_params=pltpu.CompilerParams(needs_layout_passes=False),
    )
    def body(tt_hbm, out_hbm, arena, sem0, sem1):
        w = lax.axis_index("s") * 2 + lax.axis_index("c")
        sems = (sem0, sem1)

        pltpu.sync_copy(tt_hbm, arena.at[pl.ds(0, TT_W)])

        u0 = 1984 - ROWS_PER_W * w
        lanes = lax.iota(jnp.int32, 16)

        @plsc.parallel_loop(0, EMB * (WCOLS // 16), step=1, unroll=4)
        def build(k):
            c = k // (WCOLS // 16)
            xv = k % (WCOLS // 16)
            lo = c * NPOS
            base = lo + u0 + 16 * xv - HEAD
            idx = jnp.clip(lanes + base, lo, lo + NPOS - 1)
            vals = plsc.load_gather(arena, [idx])
            arena[pl.ds(WT_OFF + c * WCOLS + 16 * xv, 16)] = vals

        first = w * ROWS_PER_W

        def emit_row(t, carry):
            delta = (ROWS_PER_W - 1) - t
            for tr in range(4):
                b = tr % 2

                def reclaim(b=b):
                    pltpu.make_async_copy(
                        arena.at[pl.ds(b * CHUNK, CHUNK)],
                        out_hbm.at[pl.ds(0, CHUNK)],
                        sems[b],
                    ).wait()

                if tr < 2:
                    pl.when(t > 0)(reclaim)
                else:
                    reclaim()

                @plsc.parallel_loop(0, 128, step=1, unroll=8)
                def pack(blk):
                    tc = blk // 8
                    r = blk % 8
                    src = WT_OFF + (8 * tr + r) * WCOLS + delta + 128 * tc
                    dst = b * CHUNK + blk * 128
                    for q in range(8):
                        arena[pl.ds(dst + 16 * q, 16)] = arena[
                            pl.ds(src + 16 * q, 16)
                        ]

                out_off = pl.multiple_of(
                    (first + t) * ROW_WORDS + tr * CHUNK, CHUNK
                )
                pltpu.async_copy(
                    arena.at[pl.ds(b * CHUNK, CHUNK)],
                    out_hbm.at[pl.ds(out_off, CHUNK)],
                    sems[b],
                )
            return carry

        lax.fori_loop(0, ROWS_PER_W, emit_row, 0)

        for b in range(2):
            pltpu.make_async_copy(
                arena.at[pl.ds(b * CHUNK, CHUNK)],
                out_hbm.at[pl.ds(0, CHUNK)],
                sems[b],
            ).wait()

    return body(table_t)


def kernel(seq_len_q, seq_len_k, embeddings_table):
    del seq_len_q, seq_len_k
    flat = _sc_call(embeddings_table.T.reshape(-1))
    s = flat.reshape(SEQ, 4, 16, 8, 128)
    return s.transpose(0, 2, 4, 1, 3).reshape(SEQ, SEQ, EMB)
